# baseline (speedup 1.0000x reference)
import functools
import jax
import jax.numpy as jnp
from jax import lax
from jax.experimental import pallas as pl
from jax.experimental.pallas import tpu as pltpu
from jax.experimental.pallas import tpu_sc as plsc

_NC = 2
_NS = 16
_NW = _NC * _NS


def _deg_body(np_, nch, ch, dst_hbm, ew_hbm, deg_out, dst_v, ew_v, tmp_v,
              deg_sh):
    c = lax.axis_index("c")
    s = lax.axis_index("s")
    wid = s * _NC + c
    rpt = np_ // _NS

    z16 = jnp.zeros((16,), jnp.float32)

    @pl.loop(0, rpt // 16)
    def _zero(i):
        tmp_v[pl.ds(i * 16, 16)] = z16

    pltpu.sync_copy(tmp_v, deg_sh.at[pl.ds(s * rpt, rpt)])
    plsc.subcore_barrier()

    pltpu.sync_copy(dst_hbm.at[wid], dst_v)
    pltpu.sync_copy(ew_hbm.at[wid], ew_v)

    @pl.loop(0, nch)
    def _scatter(k):
        pltpu.sync_copy(ew_v.at[k], deg_sh.at[dst_v.at[k]], add=True)

    plsc.subcore_barrier()
    pltpu.sync_copy(deg_sh.at[pl.ds(s * rpt, rpt)], tmp_v)
    pltpu.sync_copy(tmp_v, deg_out.at[c, pl.ds(s * rpt, rpt)])


def _edge_body(np_, nch, ch, u_hbm, src_hbm, dst_hbm, ew_hbm, y_out,
               src_v, dst_v, ew_v, rows_v, tmp_v, y_sh, sem_g, sem_s):
    c = lax.axis_index("c")
    s = lax.axis_index("s")
    wid = s * _NC + c
    rpt = np_ // _NS

    z16 = jnp.zeros((16,), jnp.float32)

    @pl.loop(0, rpt)
    def _zero(i):
        tmp_v[i, pl.ds(0, 16)] = z16
        tmp_v[i, pl.ds(16, 16)] = z16

    pltpu.sync_copy(tmp_v, y_sh.at[pl.ds(s * rpt, rpt)])
    plsc.subcore_barrier()

    pltpu.sync_copy(src_hbm.at[wid], src_v)
    pltpu.sync_copy(dst_hbm.at[wid], dst_v)
    pltpu.sync_copy(ew_hbm.at[wid], ew_v)

    @pl.loop(0, nch // 8)
    def _super(kk):
        for b in range(8):
            k = kk * 8 + b
            buf = rows_v.at[b]
            pltpu.async_copy(u_hbm.at[src_v.at[k]], buf, sem_g).wait()

            @pl.loop(0, ch // 16)
            def _scale(j):
                wv = ew_v[k, pl.ds(j * 16, 16)]
                for l in range(16):
                    e = j * 16 + l
                    w = wv[l]
                    buf[e, pl.ds(0, 16)] = buf[e, pl.ds(0, 16)] * w
                    buf[e, pl.ds(16, 16)] = buf[e, pl.ds(16, 16)] * w

            pltpu.async_copy(buf, y_sh.at[dst_v.at[k]], sem_s, add=True)

        for b in range(8):
            k = kk * 8 + b
            pltpu.make_async_copy(rows_v.at[b], y_sh.at[dst_v.at[k]],
                                  sem_s).wait()

    plsc.subcore_barrier()
    pltpu.sync_copy(y_sh.at[pl.ds(s * rpt, rpt)], tmp_v)
    pltpu.sync_copy(tmp_v, y_out.at[c, pl.ds(s * rpt, rpt)])


def _scale_body(deg0, deg1, xq, dinv, u):
    d = deg0[...] + deg1[...] + 1.0
    r = lax.rsqrt(d)
    dinv[...] = r
    u[...] = xq[...] * r


def _epi_body(y0, y1, xf, dinv, att, wz, bz, lwz, lbz, wh, bh, lwh, lbh, out):
    f32 = jnp.float32
    r = dinv[...]
    ytot = (y0[...] + y1[...]) * r + xf[...] * (r * r)

    az = lwz[...][0:128, :]
    ah = lwh[...][0:128, :]
    mz = jnp.dot(wz[...], az, preferred_element_type=f32)
    mh = jnp.dot(wh[...], ah, preferred_element_type=f32)
    cz = jnp.dot(bz[...], az, preferred_element_type=f32) + lbz[...]
    chh = jnp.dot(bh[...], ah, preferred_element_type=f32) + lbh[...]

    a = att[...]
    a = a - jnp.max(a, axis=1, keepdims=True)
    e = jnp.exp(a)
    probs = e / jnp.sum(e, axis=1, keepdims=True)

    acc = jnp.zeros(out.shape, f32)
    for t in range(16):
        x0 = ytot[:, t:t + 1]
        x1 = ytot[:, 16 + t:17 + t]
        gz = x0 * mz[0:1, :] + x1 * mz[1:2, :] + cz
        gh = x0 * mh[0:1, :] + x1 * mh[1:2, :] + chh
        z = jax.nn.sigmoid(gz)
        ht = jnp.tanh(gh)
        acc = acc + probs[:, t:t + 1] * ((1.0 - z) * ht)
    out[...] = acc


def kernel(X, edge_index, edge_weight, attention, W_z, b_z, lw_z, lb_z,
           W_r, b_r, lw_r, lb_r, W_h, b_h, lw_h, lb_h):
    f32 = jnp.float32
    N = X.shape[0]
    E = edge_index.shape[1]
    P2 = X.shape[2]
    OC = lw_z.shape[1]
    NP = ((N + 2047) // 2048) * 2048

    ch = 80
    nch = -(-E // (_NW * ch))
    nch = ((nch + 7) // 8) * 8
    E2 = _NW * ch * nch

    xf = X.reshape(N, 2 * P2)
    pe = E2 - E
    src = jnp.pad(edge_index[0], (0, pe)).reshape(_NW, nch, ch)
    dst = jnp.pad(edge_index[1], (0, pe)).reshape(_NW, nch, ch)
    ew = jnp.pad(edge_weight, (0, pe)).reshape(_NW, nch, ch)

    mesh = plsc.VectorSubcoreMesh(core_axis_name="c", subcore_axis_name="s")

    deg_k = pl.kernel(
        functools.partial(_deg_body, NP, nch, ch),
        out_type=jax.ShapeDtypeStruct((_NC, NP), f32),
        mesh=mesh,
        scratch_types=[
            pltpu.VMEM((nch, ch), jnp.int32),
            pltpu.VMEM((nch, ch), f32),
            pltpu.VMEM((NP // _NS,), f32),
            pltpu.VMEM_SHARED((NP,), f32),
        ],
    )
    deg2 = deg_k(dst, ew)

    BB = 2000
    g = N // BB
    dinv, u = pl.pallas_call(
        _scale_body,
        grid=(g,),
        in_specs=[
            pl.BlockSpec((BB, 1), lambda i: (i, 0)),
            pl.BlockSpec((BB, 1), lambda i: (i, 0)),
            pl.BlockSpec((BB, 2 * P2), lambda i: (i, 0)),
        ],
        out_specs=[
            pl.BlockSpec((BB, 1), lambda i: (i, 0)),
            pl.BlockSpec((BB, 2 * P2), lambda i: (i, 0)),
        ],
        out_shape=[
            jax.ShapeDtypeStruct((N, 1), f32),
            jax.ShapeDtypeStruct((N, 2 * P2), f32),
        ],
    )(deg2[0].reshape(NP, 1), deg2[1].reshape(NP, 1), xf)

    edge_k = pl.kernel(
        functools.partial(_edge_body, NP, nch, ch),
        out_type=jax.ShapeDtypeStruct((_NC, NP, 2 * P2), f32),
        mesh=mesh,
        compiler_params=pltpu.CompilerParams(use_tc_tiling_on_sc=False,
                                             needs_layout_passes=False),
        scratch_types=[
            pltpu.VMEM((nch, ch), jnp.int32),
            pltpu.VMEM((nch, ch), jnp.int32),
            pltpu.VMEM((nch, ch), f32),
            pltpu.VMEM((8, ch, 2 * P2), f32),
            pltpu.VMEM((NP // _NS, 2 * P2), f32),
            pltpu.VMEM_SHARED((NP, 2 * P2), f32),
            pltpu.SemaphoreType.DMA,
            pltpu.SemaphoreType.DMA,
        ],
    )
    y2 = edge_k(u, src, dst, ew)

    full = lambda shape: pl.BlockSpec(shape, lambda i: tuple(0 for _ in shape))
    out = pl.pallas_call(
        _epi_body,
        grid=(g,),
        in_specs=[
            pl.BlockSpec((BB, 2 * P2), lambda i: (i, 0)),
            pl.BlockSpec((BB, 2 * P2), lambda i: (i, 0)),
            pl.BlockSpec((BB, 2 * P2), lambda i: (i, 0)),
            pl.BlockSpec((BB, 1), lambda i: (i, 0)),
            full((1, P2)),
            full((2, OC)),
            full((1, OC)),
            full((2 * OC, OC)),
            full((1, OC)),
            full((2, OC)),
            full((1, OC)),
            full((2 * OC, OC)),
            full((1, OC)),
        ],
        out_specs=pl.BlockSpec((BB, OC), lambda i: (i, 0)),
        out_shape=jax.ShapeDtypeStruct((N, OC), f32),
    )(y2[0], y2[1], xf, dinv, attention.reshape(1, P2),
      W_z, b_z.reshape(1, OC), lw_z, lb_z.reshape(1, OC),
      W_h, b_h.reshape(1, OC), lw_h, lb_h.reshape(1, OC))

    return out


# speedup vs baseline: 1.1497x; 1.1497x over previous
import functools
import jax
import jax.numpy as jnp
from jax import lax
from jax.experimental import pallas as pl
from jax.experimental.pallas import tpu as pltpu
from jax.experimental.pallas import tpu_sc as plsc

_NC = 2
_NS = 16
_NW = _NC * _NS


def _deg_body(np_, nch, ch, dst_hbm, ew_hbm, deg_out, dst_v, ew_v, tmp_v,
              deg_sh):
    c = lax.axis_index("c")
    s = lax.axis_index("s")
    wid = s * _NC + c
    rpt = np_ // _NS

    z16 = jnp.zeros((16,), jnp.float32)

    @pl.loop(0, rpt // 16)
    def _zero(i):
        tmp_v[pl.ds(i * 16, 16)] = z16

    pltpu.sync_copy(tmp_v, deg_sh.at[pl.ds(s * rpt, rpt)])
    plsc.subcore_barrier()

    pltpu.sync_copy(dst_hbm.at[wid], dst_v)
    pltpu.sync_copy(ew_hbm.at[wid], ew_v)

    @pl.loop(0, nch)
    def _scatter(k):
        pltpu.sync_copy(ew_v.at[k], deg_sh.at[dst_v.at[k]], add=True)

    plsc.subcore_barrier()
    pltpu.sync_copy(deg_sh.at[pl.ds(s * rpt, rpt)], tmp_v)
    pltpu.sync_copy(tmp_v, deg_out.at[c, pl.ds(s * rpt, rpt)])


def _edge_body(np_, nch, ch, u_hbm, src_hbm, dst_hbm, ew_hbm, y_out,
               src_v, dst_v, ew_v, rows_v, tmp_v, y_sh, sem_g):
    c = lax.axis_index("c")
    s = lax.axis_index("s")
    wid = s * _NC + c
    rpt = np_ // _NS

    z16 = jnp.zeros((16,), jnp.float32)

    @pl.loop(0, rpt)
    def _zero(i):
        tmp_v[i, pl.ds(0, 16)] = z16
        tmp_v[i, pl.ds(16, 16)] = z16

    pltpu.sync_copy(tmp_v, y_sh.at[pl.ds(s * rpt, rpt)])
    plsc.subcore_barrier()

    pltpu.sync_copy(src_hbm.at[wid], src_v)
    pltpu.sync_copy(dst_hbm.at[wid], dst_v)
    pltpu.sync_copy(ew_hbm.at[wid], ew_v)

    @pl.loop(0, nch)
    def _chunk(k):
        pltpu.async_copy(u_hbm.at[src_v.at[k]], rows_v, sem_g).wait()

        @pl.loop(0, ch // 16)
        def _scale(j):
            wv = ew_v[k, pl.ds(j * 16, 16)]
            for l in range(16):
                e = j * 16 + l
                w = wv[l]
                rows_v[e, pl.ds(0, 16)] = rows_v[e, pl.ds(0, 16)] * w
                rows_v[e, pl.ds(16, 16)] = rows_v[e, pl.ds(16, 16)] * w

        pltpu.sync_copy(rows_v, y_sh.at[dst_v.at[k]], add=True)

    plsc.subcore_barrier()
    pltpu.sync_copy(y_sh.at[pl.ds(s * rpt, rpt)], tmp_v)
    pltpu.sync_copy(tmp_v, y_out.at[c, pl.ds(s * rpt, rpt)])


def _scale_body(deg0, deg1, xq, dinv, u):
    d = deg0[...] + deg1[...] + 1.0
    r = lax.rsqrt(d)
    dinv[...] = r
    u[...] = xq[...] * r


def _epi_body(y0, y1, xf, dinv, att, wz, bz, lwz, lbz, wh, bh, lwh, lbh, out):
    f32 = jnp.float32
    r = dinv[...]
    ytot = (y0[...] + y1[...]) * r + xf[...] * (r * r)

    az = lwz[...][0:128, :]
    ah = lwh[...][0:128, :]
    mz = jnp.dot(wz[...], az, preferred_element_type=f32)
    mh = jnp.dot(wh[...], ah, preferred_element_type=f32)
    cz = jnp.dot(bz[...], az, preferred_element_type=f32) + lbz[...]
    chh = jnp.dot(bh[...], ah, preferred_element_type=f32) + lbh[...]

    a = att[...]
    a = a - jnp.max(a, axis=1, keepdims=True)
    e = jnp.exp(a)
    probs = e / jnp.sum(e, axis=1, keepdims=True)

    acc = jnp.zeros(out.shape, f32)
    for t in range(16):
        x0 = ytot[:, t:t + 1]
        x1 = ytot[:, 16 + t:17 + t]
        gz = x0 * mz[0:1, :] + x1 * mz[1:2, :] + cz
        gh = x0 * mh[0:1, :] + x1 * mh[1:2, :] + chh
        z = jax.nn.sigmoid(gz)
        ht = jnp.tanh(gh)
        acc = acc + probs[:, t:t + 1] * ((1.0 - z) * ht)
    out[...] = acc


def kernel(X, edge_index, edge_weight, attention, W_z, b_z, lw_z, lb_z,
           W_r, b_r, lw_r, lb_r, W_h, b_h, lw_h, lb_h):
    f32 = jnp.float32
    N = X.shape[0]
    E = edge_index.shape[1]
    P2 = X.shape[2]
    OC = lw_z.shape[1]
    NP = ((N + 2047) // 2048) * 2048

    ch = 80
    nch = -(-E // (_NW * ch))
    E2 = _NW * ch * nch

    xf = X.reshape(N, 2 * P2)
    pe = E2 - E
    src = jnp.pad(edge_index[0], (0, pe)).reshape(_NW, nch, ch)
    dst = jnp.pad(edge_index[1], (0, pe)).reshape(_NW, nch, ch)
    ew = jnp.pad(edge_weight, (0, pe)).reshape(_NW, nch, ch)

    mesh = plsc.VectorSubcoreMesh(core_axis_name="c", subcore_axis_name="s")

    deg_k = pl.kernel(
        functools.partial(_deg_body, NP, nch, ch),
        out_type=jax.ShapeDtypeStruct((_NC, NP), f32),
        mesh=mesh,
        scratch_types=[
            pltpu.VMEM((nch, ch), jnp.int32),
            pltpu.VMEM((nch, ch), f32),
            pltpu.VMEM((NP // _NS,), f32),
            pltpu.VMEM_SHARED((NP,), f32),
        ],
    )
    deg2 = deg_k(dst, ew)

    BB = 2000
    g = N // BB
    dinv, u = pl.pallas_call(
        _scale_body,
        grid=(g,),
        in_specs=[
            pl.BlockSpec((BB, 1), lambda i: (i, 0)),
            pl.BlockSpec((BB, 1), lambda i: (i, 0)),
            pl.BlockSpec((BB, 2 * P2), lambda i: (i, 0)),
        ],
        out_specs=[
            pl.BlockSpec((BB, 1), lambda i: (i, 0)),
            pl.BlockSpec((BB, 2 * P2), lambda i: (i, 0)),
        ],
        out_shape=[
            jax.ShapeDtypeStruct((N, 1), f32),
            jax.ShapeDtypeStruct((N, 2 * P2), f32),
        ],
    )(deg2[0].reshape(NP, 1), deg2[1].reshape(NP, 1), xf)

    edge_k = pl.kernel(
        functools.partial(_edge_body, NP, nch, ch),
        out_type=jax.ShapeDtypeStruct((_NC, NP, 2 * P2), f32),
        mesh=mesh,
        compiler_params=pltpu.CompilerParams(use_tc_tiling_on_sc=False),
        scratch_types=[
            pltpu.VMEM((nch, ch), jnp.int32),
            pltpu.VMEM((nch, ch), jnp.int32),
            pltpu.VMEM((nch, ch), f32),
            pltpu.VMEM((ch, 2 * P2), f32),
            pltpu.VMEM((NP // _NS, 2 * P2), f32),
            pltpu.VMEM_SHARED((NP, 2 * P2), f32),
            pltpu.SemaphoreType.DMA,
        ],
    )
    y2 = edge_k(u, src, dst, ew)

    full = lambda shape: pl.BlockSpec(shape, lambda i: tuple(0 for _ in shape))
    out = pl.pallas_call(
        _epi_body,
        grid=(g,),
        in_specs=[
            pl.BlockSpec((BB, 2 * P2), lambda i: (i, 0)),
            pl.BlockSpec((BB, 2 * P2), lambda i: (i, 0)),
            pl.BlockSpec((BB, 2 * P2), lambda i: (i, 0)),
            pl.BlockSpec((BB, 1), lambda i: (i, 0)),
            full((1, P2)),
            full((2, OC)),
            full((1, OC)),
            full((2 * OC, OC)),
            full((1, OC)),
            full((2, OC)),
            full((1, OC)),
            full((2 * OC, OC)),
            full((1, OC)),
        ],
        out_specs=pl.BlockSpec((BB, OC), lambda i: (i, 0)),
        out_shape=jax.ShapeDtypeStruct((N, OC), f32),
    )(y2[0], y2[1], xf, dinv, attention.reshape(1, P2),
      W_z, b_z.reshape(1, OC), lw_z, lb_z.reshape(1, OC),
      W_h, b_h.reshape(1, OC), lw_h, lb_h.reshape(1, OC))

    return out
